# baseline (speedup 1.0000x reference)
import functools

import jax
import jax.numpy as jnp
from jax import lax
from jax.experimental import pallas as pl
from jax.experimental.pallas import tpu as pltpu
from jax.experimental.pallas import tpu_sc as plsc

VOCAB_SIZE = 1000
EMBED_DIM = 128
BATCH = 4096
SEQ = 200

NC = 2
NS = 16
NW = NC * NS

B = BATCH * SEQ
B_PER_W = B // NW
ROWS = 128
NCH = B_PER_W // ROWS
NSLOT = 5
F = 3
HBM_SLOTS = frozenset({0})


def _make_kernel():
    mesh = plsc.VectorSubcoreMesh(
        core_axis_name="c", subcore_axis_name="s",
        num_cores=NC, num_subcores=NS)

    @functools.partial(
        pl.kernel,
        mesh=mesh,
        out_type=jax.ShapeDtypeStruct((B, EMBED_DIM), jnp.float32),
        scratch_types=[
            pltpu.VMEM((NCH, ROWS), jnp.int32),
            pltpu.VMEM((NSLOT, ROWS, EMBED_DIM), jnp.float32),
            pltpu.VMEM_SHARED((VOCAB_SIZE, EMBED_DIM), jnp.float32),
            [pltpu.SemaphoreType.DMA] * NSLOT,
            [pltpu.SemaphoreType.DMA] * NSLOT,
        ],
    )
    def emb_kernel(x_hbm, w_hbm, out_hbm, idx_v, rows_v, w_sh, gsems, osems):
        sid = lax.axis_index("s")
        wid = sid * NC + lax.axis_index("c")
        base = wid * B_PER_W

        @pl.when(sid == 0)
        def _stage_table():
            pltpu.sync_copy(w_hbm, w_sh)

        pltpu.sync_copy(x_hbm.at[wid], idx_v)
        plsc.subcore_barrier()

        def _gsrc(u):
            return w_hbm if u in HBM_SLOTS else w_sh

        def fire_g(c, u):
            pltpu.async_copy(
                _gsrc(u).at[idx_v.at[c]], rows_v.at[u], gsems[u])

        def drain_g(c, u):
            pltpu.make_async_copy(
                _gsrc(u).at[idx_v.at[c]], rows_v.at[u], gsems[u]).wait()

        def fire_s(c, u):
            pltpu.async_copy(
                rows_v.at[u], out_hbm.at[pl.ds(base + c * ROWS, ROWS)],
                osems[u])

        def drain_s(c, u):
            pltpu.make_async_copy(
                rows_v.at[u], out_hbm.at[pl.ds(base + c * ROWS, ROWS)],
                osems[u]).wait()

        def emit(c, u, drain_old=True, fire_new=True):
            drain_g(c, u)
            fire_s(c, u)
            ju = (u + F) % NSLOT
            if drain_old:
                drain_s(c + F - NSLOT, ju)
            if fire_new:
                fire_g(c + F, ju)

        for j in range(F):
            fire_g(j, j)
        for c in range(NSLOT - F):
            emit(c, c % NSLOT, drain_old=False)
        for c in range(NSLOT - F, NSLOT):
            emit(c, c % NSLOT)

        top = ((NCH - F - 1) // NSLOT) * NSLOT

        def body(m, carry):
            for u in range(NSLOT):
                emit(m * NSLOT + u, u)
            return carry

        lax.fori_loop(1, top // NSLOT, body, 0)

        for c in range(top, NCH - F):
            emit(c, c % NSLOT)
        for c in range(NCH - F, NCH):
            emit(c, c % NSLOT, fire_new=False)
        for c in range(NCH - (NSLOT - F), NCH):
            drain_s(c, c % NSLOT)

    return emb_kernel


_emb = _make_kernel()


def kernel(x, weight):
    x3 = x.reshape(NW, NCH, ROWS)
    out = _emb(x3, weight)
    return out.reshape(BATCH, SEQ, EMBED_DIM)


# speedup vs baseline: 1.3057x; 1.3057x over previous
import functools

import jax
import jax.numpy as jnp
from jax import lax
from jax.experimental import pallas as pl
from jax.experimental.pallas import tpu as pltpu
from jax.experimental.pallas import tpu_sc as plsc

VOCAB_SIZE = 1000
EMBED_DIM = 128
BATCH = 4096
SEQ = 200

NC = 2
NS = 16
NW = NC * NS

ROWS_PER_W = BATCH // NW
B = BATCH * SEQ
B_PER_W = ROWS_PER_W * SEQ
NG = ROWS_PER_W
QG = 32
NQ = NG // QG
NSLOT = 4
F = 2
SPLITS = (0, 128, SEQ)


def _make_kernel():
    mesh = plsc.VectorSubcoreMesh(
        core_axis_name="c", subcore_axis_name="s",
        num_cores=NC, num_subcores=NS)

    @functools.partial(
        pl.kernel,
        mesh=mesh,
        out_type=jax.ShapeDtypeStruct((B, EMBED_DIM), jnp.float32),
        scratch_types=[
            pltpu.VMEM((2, QG, SEQ), jnp.int32),
            pltpu.VMEM((NSLOT, SEQ, EMBED_DIM), jnp.float32),
            pltpu.VMEM_SHARED((VOCAB_SIZE, EMBED_DIM), jnp.float32),
            [pltpu.SemaphoreType.DMA] * 2,
            [pltpu.SemaphoreType.DMA] * NSLOT,
            [pltpu.SemaphoreType.DMA] * NSLOT,
        ],
    )
    def emb_kernel(x_hbm, w_hbm, out_hbm, idx_v, rows_v, w_sh,
                   isems, gsems, osems):
        sid = lax.axis_index("s")
        wid = sid * NC + lax.axis_index("c")
        base = wid * B_PER_W
        xrow = wid * ROWS_PER_W

        def fire_i(q):
            b = q % 2
            pltpu.async_copy(
                x_hbm.at[pl.ds(xrow + q * QG, QG)], idx_v.at[b], isems[b])

        def drain_i(q):
            b = q % 2
            pltpu.make_async_copy(
                x_hbm.at[pl.ds(xrow + q * QG, QG)], idx_v.at[b],
                isems[b]).wait()

        fire_i(0)
        fire_i(1)

        @pl.when(sid == 0)
        def _stage_table():
            pltpu.sync_copy(w_hbm, w_sh)
        plsc.subcore_barrier()

        def _gcopies(b, r, u):
            for t in range(len(SPLITS) - 1):
                lo, hi = SPLITS[t], SPLITS[t + 1]
                yield (w_sh.at[idx_v.at[b, r, pl.ds(lo, hi - lo)]],
                       rows_v.at[u, pl.ds(lo, hi - lo)], gsems[u])

        def fire_g(b, r, u):
            for args in _gcopies(b, r, u):
                pltpu.async_copy(*args)

        def drain_g(b, r, u):
            for args in _gcopies(b, r, u):
                pltpu.make_async_copy(*args).wait()

        def fire_s(g, u):
            pltpu.async_copy(
                rows_v.at[u], out_hbm.at[pl.ds(base + g * SEQ, SEQ)],
                osems[u])

        def drain_s(g, u):
            pltpu.make_async_copy(
                rows_v.at[u], out_hbm.at[pl.ds(base + g * SEQ, SEQ)],
                osems[u]).wait()

        def emit(p, lc, u, fire=None, drain_old=True):
            g = p * QG + lc
            drain_g(p % 2, lc, u)
            fire_s(g, u)
            ju = (u + F) % NSLOT
            if drain_old:
                drain_s(g - (NSLOT - F), ju)
            if fire is not None:
                pf, lf = fire
                fire_g(pf % 2, lf, ju)

        drain_i(0)
        fire_g(0, 0, 0)
        fire_g(0, 1, 1)
        emit(0, 0, 0, fire=(0, 2), drain_old=False)
        emit(0, 1, 1, fire=(0, 3), drain_old=False)
        emit(0, 2, 2, fire=(0, 4))
        emit(0, 3, 3, fire=(0, 5))

        def make_body(p):
            def body(m, carry):
                lc = m * NSLOT
                for u in range(NSLOT):
                    emit(p, lc + u, u, fire=(p, lc + u + F))
                return carry
            return body

        lax.fori_loop(1, 7, make_body(0), 0)
        emit(0, 28, 0, fire=(0, 30))
        emit(0, 29, 1, fire=(0, 31))
        drain_i(1)
        emit(0, 30, 2, fire=(1, 0))
        emit(0, 31, 3, fire=(1, 1))

        for p in (1, 2):
            fire_i(p + 1)
            lax.fori_loop(0, 7, make_body(p), 0)
            emit(p, 28, 0, fire=(p, 30))
            emit(p, 29, 1, fire=(p, 31))
            drain_i(p + 1)
            emit(p, 30, 2, fire=(p + 1, 0))
            emit(p, 31, 3, fire=(p + 1, 1))

        lax.fori_loop(0, 7, make_body(3), 0)
        emit(3, 28, 0, fire=(3, 30))
        emit(3, 29, 1, fire=(3, 31))
        emit(3, 30, 2, fire=None)
        emit(3, 31, 3, fire=None)
        drain_s(NG - 2, (NG - 2) % NSLOT)
        drain_s(NG - 1, (NG - 1) % NSLOT)

    return emb_kernel


_emb = _make_kernel()


def kernel(x, weight):
    out = _emb(x, weight)
    return out.reshape(BATCH, SEQ, EMBED_DIM)
